# baseline (speedup 1.0000x reference)
import functools
import jax
import jax.numpy as jnp
from jax import lax
from jax.experimental import pallas as pl
from jax.experimental.pallas import tpu as pltpu
from jax.experimental.pallas import tpu_sc as plsc

BATCH = 16384
INPUT_SIZE = 26
EMBED_DIM = 16

NUM_WORKERS = 32
ROWS_PER_WORKER = BATCH // NUM_WORKERS
CHUNK_ROWS = 64
NUM_CHUNKS = ROWS_PER_WORKER // CHUNK_ROWS


def _make_kernel():
    info = plsc.get_sparse_core_info()
    nc = info.num_cores

    mesh = plsc.VectorSubcoreMesh(core_axis_name="c", subcore_axis_name="s")

    @functools.partial(
        pl.kernel,
        out_type=jax.ShapeDtypeStruct((BATCH,), jnp.float32),
        mesh=mesh,
        scratch_types=[
            pltpu.VMEM((ROWS_PER_WORKER, INPUT_SIZE), jnp.int32),
            pltpu.VMEM((INPUT_SIZE, ROWS_PER_WORKER), jnp.int32),
            pltpu.VMEM((INPUT_SIZE, CHUNK_ROWS, EMBED_DIM), jnp.float32),
            pltpu.VMEM((INPUT_SIZE, CHUNK_ROWS, EMBED_DIM), jnp.float32),
            pltpu.VMEM((INPUT_SIZE, EMBED_DIM), jnp.float32),
            pltpu.VMEM((EMBED_DIM,), jnp.float32),
            pltpu.VMEM((ROWS_PER_WORKER,), jnp.float32),
            pltpu.SemaphoreType.DMA,
            pltpu.SemaphoreType.DMA,
        ],
        compiler_params=pltpu.CompilerParams(
            needs_layout_passes=False, use_tc_tiling_on_sc=False),
    )
    def emb_kernel(x_hbm, w_hbm, b_hbm, table_hbm, y_hbm,
                   slab_v, idx_v, rows_a, rows_b, w_v, b_v, y_v,
                   sem_a, sem_b):
        wid = lax.axis_index("s") * nc + lax.axis_index("c")

        row0 = wid * ROWS_PER_WORKER
        pltpu.sync_copy(x_hbm.at[pl.ds(row0, ROWS_PER_WORKER), :], slab_v)

        lane16 = lax.iota(jnp.int32, EMBED_DIM)

        def transpose_j(j, _):
            def transpose_g(g, _):
                v = plsc.load_gather(
                    slab_v, [g * 16 + lane16, jnp.full((16,), j, jnp.int32)])
                idx_v[j, pl.ds(g * 16, 16)] = v
                return 0
            lax.fori_loop(0, ROWS_PER_WORKER // 16, transpose_g, 0)
            return 0

        lax.fori_loop(0, INPUT_SIZE, transpose_j, 0)
        pltpu.sync_copy(w_hbm, w_v)
        pltpu.sync_copy(b_hbm, b_v)

        def gather(buf, c, sem):
            descs = []
            for j in range(INPUT_SIZE):
                descs.append(pltpu.async_copy(
                    table_hbm.at[idx_v.at[j, pl.ds(c * CHUNK_ROWS,
                                                   CHUNK_ROWS)]],
                    buf.at[j],
                    sem,
                ))
            return descs

        lane = lax.iota(jnp.int32, EMBED_DIM)

        def compute(buf, c):
            bias = b_v[:][0]

            def group_body(g, _):
                def row_body(rr, yvec):
                    r = g * 16 + rr
                    acc = buf[0, r, :] * w_v[0, :]
                    for j in range(1, INPUT_SIZE):
                        acc = acc + buf[j, r, :] * w_v[j, :]
                    val = jnp.sum(acc) + bias
                    return jnp.where(lane == rr, val, yvec)

                yvec = lax.fori_loop(
                    0, 16, row_body, jnp.zeros((EMBED_DIM,), jnp.float32))
                y_v[pl.ds(c * CHUNK_ROWS + g * 16, 16)] = yvec
                return 0

            lax.fori_loop(0, CHUNK_ROWS // 16, group_body, 0)

        bufs = (rows_a, rows_b)
        sems = (sem_a, sem_b)
        pending = gather(bufs[0], 0, sems[0])
        for c in range(NUM_CHUNKS):
            for d in pending:
                d.wait()
            if c + 1 < NUM_CHUNKS:
                pending = gather(bufs[(c + 1) % 2], c + 1, sems[(c + 1) % 2])
            compute(bufs[c % 2], c)

        pltpu.sync_copy(y_v, y_hbm.at[pl.ds(wid * ROWS_PER_WORKER,
                                            ROWS_PER_WORKER)])

    return emb_kernel


_EMB_KERNEL = _make_kernel()


@jax.jit
def kernel(X, table, W, b):
    w2 = W.reshape(INPUT_SIZE, EMBED_DIM)
    b16 = jnp.broadcast_to(b, (EMBED_DIM,)).astype(jnp.float32)
    y = _EMB_KERNEL(X.astype(jnp.int32), w2, b16, table)
    return y.reshape(BATCH, 1)


# speedup vs baseline: 1.0401x; 1.0401x over previous
import functools
import jax
import jax.numpy as jnp
from jax import lax
from jax.experimental import pallas as pl
from jax.experimental.pallas import tpu as pltpu
from jax.experimental.pallas import tpu_sc as plsc

BATCH = 16384
INPUT_SIZE = 26
EMBED_DIM = 16

NUM_WORKERS = 32
ROWS_PER_WORKER = BATCH // NUM_WORKERS
CHUNK_ROWS = 128
NUM_CHUNKS = ROWS_PER_WORKER // CHUNK_ROWS


def _make_detile():
    info = plsc.get_sparse_core_info()
    nc = info.num_cores
    mesh = plsc.VectorSubcoreMesh(core_axis_name="c", subcore_axis_name="s")

    @functools.partial(
        pl.kernel,
        out_type=jax.ShapeDtypeStruct((INPUT_SIZE * BATCH,), jnp.int32),
        mesh=mesh,
        scratch_types=[
            pltpu.VMEM((8, ROWS_PER_WORKER), jnp.int32),
        ],
        compiler_params=pltpu.CompilerParams(
            needs_layout_passes=False, use_tc_tiling_on_sc=True),
    )
    def detile(xt_hbm, out_hbm, v):
        wid = lax.axis_index("s") * nc + lax.axis_index("c")
        col0 = wid * ROWS_PER_WORKER
        for t in range((INPUT_SIZE + 7) // 8):
            nr = min(8, INPUT_SIZE - t * 8)
            pltpu.sync_copy(
                xt_hbm.at[pl.ds(t * 8, nr), pl.ds(col0, ROWS_PER_WORKER)],
                v.at[pl.ds(0, nr)])
            for r in range(nr):
                j = t * 8 + r
                pltpu.sync_copy(
                    v.at[r],
                    out_hbm.at[pl.ds(j * BATCH + col0, ROWS_PER_WORKER)])

    return detile


def _make_kernel():
    info = plsc.get_sparse_core_info()
    nc = info.num_cores
    mesh = plsc.VectorSubcoreMesh(core_axis_name="c", subcore_axis_name="s")

    @functools.partial(
        pl.kernel,
        out_type=jax.ShapeDtypeStruct((BATCH,), jnp.float32),
        mesh=mesh,
        scratch_types=[
            pltpu.VMEM((INPUT_SIZE, ROWS_PER_WORKER), jnp.int32),
            pltpu.VMEM((INPUT_SIZE, CHUNK_ROWS, EMBED_DIM), jnp.float32),
            pltpu.VMEM((INPUT_SIZE, CHUNK_ROWS, EMBED_DIM), jnp.float32),
            pltpu.VMEM((INPUT_SIZE, EMBED_DIM), jnp.float32),
            pltpu.VMEM((EMBED_DIM,), jnp.float32),
            pltpu.VMEM((ROWS_PER_WORKER,), jnp.float32),
            pltpu.SemaphoreType.DMA,
            pltpu.SemaphoreType.DMA,
        ],
        compiler_params=pltpu.CompilerParams(
            needs_layout_passes=False, use_tc_tiling_on_sc=False),
    )
    def emb_kernel(x1_hbm, w_hbm, b_hbm, table_hbm, y_hbm,
                   idx_v, rows_a, rows_b, w_v, b_v, y_v, sem_a, sem_b):
        wid = lax.axis_index("s") * nc + lax.axis_index("c")
        row0 = wid * ROWS_PER_WORKER

        idx_descs = [
            pltpu.async_copy(
                x1_hbm.at[pl.ds(j * BATCH + row0, ROWS_PER_WORKER)],
                idx_v.at[j],
                sem_a,
            )
            for j in range(INPUT_SIZE)
        ]
        pltpu.sync_copy(w_hbm, w_v)
        pltpu.sync_copy(b_hbm, b_v)
        for d in idx_descs:
            d.wait()

        def gather(buf, c, sem):
            descs = []
            for j in range(INPUT_SIZE):
                descs.append(pltpu.async_copy(
                    table_hbm.at[idx_v.at[j, pl.ds(c * CHUNK_ROWS,
                                                   CHUNK_ROWS)]],
                    buf.at[j],
                    sem,
                ))
            return descs

        lane = lax.iota(jnp.int32, EMBED_DIM)

        def compute(buf, c):
            bias = b_v[:][0]

            def group_body(g, _):
                def row_body(rr, yvec):
                    r = g * 16 + rr
                    acc = buf[0, r, :] * w_v[0, :]
                    for j in range(1, INPUT_SIZE):
                        acc = acc + buf[j, r, :] * w_v[j, :]
                    val = jnp.sum(acc) + bias
                    return jnp.where(lane == rr, val, yvec)

                yvec = lax.fori_loop(
                    0, 16, row_body, jnp.zeros((EMBED_DIM,), jnp.float32))
                y_v[pl.ds(c * CHUNK_ROWS + g * 16, 16)] = yvec
                return 0

            lax.fori_loop(0, CHUNK_ROWS // 16, group_body, 0)

        bufs = (rows_a, rows_b)
        sems = (sem_a, sem_b)
        pending = gather(bufs[0], 0, sems[0])
        for c in range(NUM_CHUNKS):
            for d in pending:
                d.wait()
            if c + 1 < NUM_CHUNKS:
                pending = gather(bufs[(c + 1) % 2], c + 1, sems[(c + 1) % 2])
            compute(bufs[c % 2], c)

        pltpu.sync_copy(y_v, y_hbm.at[pl.ds(row0, ROWS_PER_WORKER)])

    return emb_kernel


_DETILE = _make_detile()
_EMB_KERNEL = _make_kernel()


@jax.jit
def kernel(X, table, W, b):
    x1 = _DETILE(X.T.astype(jnp.int32))
    w2 = W.reshape(INPUT_SIZE, EMBED_DIM)
    b16 = jnp.broadcast_to(b, (EMBED_DIM,)).astype(jnp.float32)
    y = _EMB_KERNEL(x1, w2, b16, table)
    return y.reshape(BATCH, 1)
